# baseline (speedup 1.0000x reference)
import functools

import jax
import jax.numpy as jnp
from jax import lax
from jax.experimental import pallas as pl
from jax.experimental.pallas import tpu as pltpu
from jax.experimental.pallas import tpu_sc as plsc

_B, _T, _DIN, _D, _K = 16, 512, 768, 64, 8192
_N = _B * _T
_TB = 256
_KB = 2048
_CHUNK = 2048
_G1 = _N // _TB


def _proj_argmin_body(inp_ref, w_ref, b_ref, emb_ref, idx_ref, md_ref):
    x = lax.dot_general(inp_ref[...].astype(jnp.bfloat16),
                        w_ref[...].astype(jnp.bfloat16),
                        (((1,), (1,)), ((), ())),
                        preferred_element_type=jnp.float32)
    x = x + b_ref[...]
    xsq = jnp.sum(x * x, axis=1, keepdims=True)
    x2b = (2.0 * x).astype(jnp.bfloat16)
    iota = lax.broadcasted_iota(jnp.int32, (_TB, _KB), 1)
    acc_v = None
    acc_i = None
    md = None
    for j in range(_K // _KB):
        e = emb_ref[j * _KB:(j + 1) * _KB, :]
        esq = jnp.sum(e * e, axis=1)
        m = lax.dot_general(x2b, e.astype(jnp.bfloat16),
                            (((1,), (1,)), ((), ())),
                            preferred_element_type=jnp.float32)
        d = (xsq + esq[None, :]) - m
        bmin = jnp.min(d, axis=1)
        bidx = jnp.min(jnp.where(d == bmin[:, None], iota, _K), axis=1) + (j * _KB)
        if acc_v is None:
            acc_v, acc_i, md = bmin, bidx, bmin
        else:
            upd = bmin < acc_v
            acc_v = jnp.where(upd, bmin, acc_v)
            acc_i = jnp.where(upd, bidx, acc_i)
            md = jnp.where(upd, bmin, md)
        if (j + 1) % (_CHUNK // _KB) == 0:
            acc_v = acc_v.astype(jnp.bfloat16).astype(jnp.float32)
    idx_ref[0, 0, :] = acc_i
    md_ref[0, 0, :] = md


def _run_proj_argmin(inp2d, w, b2d, emb):
    return pl.pallas_call(
        _proj_argmin_body,
        grid=(_G1,),
        in_specs=[
            pl.BlockSpec((_TB, _DIN), lambda i: (i, 0)),
            pl.BlockSpec((_D, _DIN), lambda i: (0, 0)),
            pl.BlockSpec((1, _D), lambda i: (0, 0)),
            pl.BlockSpec((_K, _D), lambda i: (0, 0)),
        ],
        out_specs=[
            pl.BlockSpec((1, 1, _TB), lambda i: (i, 0, 0)),
            pl.BlockSpec((1, 1, _TB), lambda i: (i, 0, 0)),
        ],
        out_shape=[
            jax.ShapeDtypeStruct((_G1, 1, _TB), jnp.int32),
            jax.ShapeDtypeStruct((_G1, 1, _TB), jnp.float32),
        ],
    )(inp2d, w, b2d, emb)


try:
    _INFO = plsc.get_sparse_core_info()
    _NC, _NS, _L = _INFO.num_cores, _INFO.num_subcores, _INFO.num_lanes
except ValueError:
    _NC, _NS, _L = 2, 16, 16
_NW = _NC * _NS
_BPW = _N // _NW
_HC = 128
_NCH = _BPW // _HC
_DP = 128
_CW = 128
_ZR = _K // _NS


def _sc_gather_hist(emb_hbm, idx_hbm, zeros_hbm, ones_hbm, q_hbm, cnt_hbm,
                    idx_v, rows_v, ones_v, shared, sem):
    c = lax.axis_index("c")
    s = lax.axis_index("s")
    wid = s * _NC + c
    base = wid * _BPW
    cps = []
    for j in range(_NCH):
        pltpu.sync_copy(idx_hbm.at[pl.ds(base + j * _HC, _HC)], idx_v.at[j])
        cps.append(pltpu.async_copy(emb_hbm.at[idx_v.at[j]], rows_v[j], sem))
    pltpu.sync_copy(ones_hbm, ones_v)
    pltpu.sync_copy(zeros_hbm, shared.at[pl.ds(s * _ZR, _ZR)])
    plsc.subcore_barrier()
    for j in range(_NCH):
        pltpu.sync_copy(ones_v, shared.at[idx_v.at[j]], add=True)
    for j in range(_NCH):
        cps[j].wait()
        pltpu.sync_copy(rows_v[j], q_hbm.at[pl.ds(base + j * _HC, _HC)])
    plsc.subcore_barrier()
    pltpu.sync_copy(shared.at[pl.ds(s * _ZR, _ZR)],
                    cnt_hbm.at[c, pl.ds(s * _ZR, _ZR)])


@functools.cache
def _get_sc_call():
    return functools.partial(
        pl.kernel,
        out_type=[
            jax.ShapeDtypeStruct((_N, _DP), jnp.float32),
            jax.ShapeDtypeStruct((_NC, _K, _CW), jnp.float32),
        ],
        mesh=plsc.VectorSubcoreMesh(core_axis_name="c", subcore_axis_name="s",
                                    num_cores=_NC, num_subcores=_NS),
        scratch_types=[
            pltpu.VMEM((_NCH, _HC), jnp.int32),
            [pltpu.VMEM((_HC, _DP), jnp.float32) for _ in range(_NCH)],
            pltpu.VMEM((_HC, _CW), jnp.float32),
            pltpu.VMEM_SHARED((_K, _CW), jnp.float32),
            pltpu.SemaphoreType.DMA,
        ],
    )(_sc_gather_hist)


def _scalar_body(cnt_ref, md_ref, vq_ref, ppl_ref):
    csum = jnp.sum(cnt_ref[...], axis=0)
    p = csum * (1.0 / _N)
    ent = -jnp.sum(p * jnp.log(p + 1e-10))
    ppl_ref[0, 0] = jnp.exp(ent)
    sl = jnp.sum(md_ref[...])
    cl = sl * (1.0 / (_N * _D))
    vq_ref[0, 0] = cl + 0.25 * cl


def _run_scalars(cnt4, md2):
    return pl.pallas_call(
        _scalar_body,
        out_specs=[
            pl.BlockSpec(memory_space=pltpu.SMEM),
            pl.BlockSpec(memory_space=pltpu.SMEM),
        ],
        out_shape=[
            jax.ShapeDtypeStruct((1, 1), jnp.float32),
            jax.ShapeDtypeStruct((1, 1), jnp.float32),
        ],
    )(cnt4, md2)


def kernel(inputs, W_proj, b_proj, emb):
    inp2d = inputs.reshape(_N, _DIN)
    b2d = b_proj.reshape(1, _D)
    idx3, md3 = _run_proj_argmin(inp2d, W_proj, b2d, emb)
    idx1d = idx3.reshape(_N)
    embp = jnp.concatenate(
        [emb, jnp.zeros((_K, _DP - _D), jnp.float32)], axis=1)
    zeros_hbm = jnp.zeros((_ZR, _CW), jnp.float32)
    ones_hbm = jnp.ones((_HC, _CW), jnp.float32)
    qp, cnt = _get_sc_call()(embp, idx1d, zeros_hbm, ones_hbm)
    q2d = qp[:, :_D]
    cnt4 = cnt[:, :, 0].reshape(_NC, _K // 128, 128)
    md2 = md3.reshape(_N // 128, 128)
    vq, ppl = _run_scalars(cnt4, md2)
    quantized_st = q2d.reshape(_B, _T, _D)
    tokens = idx1d.reshape(_B, _T)
    return quantized_st, tokens, vq[0, 0], ppl[0, 0]


# speedup vs baseline: 1.1401x; 1.1401x over previous
import functools

import jax
import jax.numpy as jnp
from jax import lax
from jax.experimental import pallas as pl
from jax.experimental.pallas import tpu as pltpu
from jax.experimental.pallas import tpu_sc as plsc

_B, _T, _DIN, _D, _K = 16, 512, 768, 64, 8192
_N = _B * _T
_TB = 512
_KB = 2048
_CHUNK = 2048
_G1 = _N // _TB


def _proj_argmin_body(inp_ref, w_ref, b_ref, emb_ref, idx_ref, md_ref):
    x = lax.dot_general(inp_ref[...].astype(jnp.bfloat16),
                        w_ref[...].astype(jnp.bfloat16),
                        (((1,), (1,)), ((), ())),
                        preferred_element_type=jnp.float32)
    x = x + b_ref[...]
    xsq = jnp.sum(x * x, axis=1, keepdims=True)
    x2b = (2.0 * x).astype(jnp.bfloat16)
    iota = lax.broadcasted_iota(jnp.int32, (_TB, _KB), 1)
    acc_v = None
    acc_i = None
    md = None
    for j in range(_K // _KB):
        e = emb_ref[j * _KB:(j + 1) * _KB, :]
        esq = jnp.sum(e * e, axis=1)
        m = lax.dot_general(x2b, e.astype(jnp.bfloat16),
                            (((1,), (1,)), ((), ())),
                            preferred_element_type=jnp.float32)
        d = (xsq + esq[None, :]) - m
        bmin = jnp.min(d, axis=1)
        bidx = jnp.min(jnp.where(d == bmin[:, None], iota, _K), axis=1) + (j * _KB)
        if acc_v is None:
            acc_v, acc_i, md = bmin, bidx, bmin
        else:
            upd = bmin < acc_v
            acc_v = jnp.where(upd, bmin, acc_v)
            acc_i = jnp.where(upd, bidx, acc_i)
            md = jnp.where(upd, bmin, md)
        if (j + 1) % (_CHUNK // _KB) == 0:
            acc_v = acc_v.astype(jnp.bfloat16).astype(jnp.float32)
    idx_ref[0, 0, :] = acc_i
    md_ref[0, 0, :] = md


def _run_proj_argmin(inp2d, w, b2d, emb):
    return pl.pallas_call(
        _proj_argmin_body,
        grid=(_G1,),
        in_specs=[
            pl.BlockSpec((_TB, _DIN), lambda i: (i, 0)),
            pl.BlockSpec((_D, _DIN), lambda i: (0, 0)),
            pl.BlockSpec((1, _D), lambda i: (0, 0)),
            pl.BlockSpec((_K, _D), lambda i: (0, 0)),
        ],
        out_specs=[
            pl.BlockSpec((1, 1, _TB), lambda i: (i, 0, 0)),
            pl.BlockSpec((1, 1, _TB), lambda i: (i, 0, 0)),
        ],
        out_shape=[
            jax.ShapeDtypeStruct((_G1, 1, _TB), jnp.int32),
            jax.ShapeDtypeStruct((_G1, 1, _TB), jnp.float32),
        ],
    )(inp2d, w, b2d, emb)


try:
    _INFO = plsc.get_sparse_core_info()
    _NC, _NS, _L = _INFO.num_cores, _INFO.num_subcores, _INFO.num_lanes
except ValueError:
    _NC, _NS, _L = 2, 16, 16
_NW = _NC * _NS
_BPW = _N // _NW
_HC = 128
_NCH = _BPW // _HC
_DP = 128
_CW = 128
_ZR = _K // _NS


def _sc_gather_hist(emb_hbm, idx_hbm, zeros_hbm, ones_hbm, q_hbm, cnt_hbm,
                    idx_v, rows_v, ones_v, shared, sem):
    c = lax.axis_index("c")
    s = lax.axis_index("s")
    wid = s * _NC + c
    base = wid * _BPW
    cps = []
    for j in range(_NCH):
        pltpu.sync_copy(idx_hbm.at[pl.ds(base + j * _HC, _HC)], idx_v.at[j])
        cps.append(pltpu.async_copy(emb_hbm.at[idx_v.at[j]], rows_v[j], sem))
    pltpu.sync_copy(ones_hbm, ones_v)
    pltpu.sync_copy(zeros_hbm, shared.at[pl.ds(s * _ZR, _ZR)])
    plsc.subcore_barrier()
    for j in range(_NCH):
        pltpu.sync_copy(ones_v, shared.at[idx_v.at[j]], add=True)
    for j in range(_NCH):
        cps[j].wait()
        pltpu.sync_copy(rows_v[j], q_hbm.at[pl.ds(base + j * _HC, _HC)])
    plsc.subcore_barrier()
    pltpu.sync_copy(shared.at[pl.ds(s * _ZR, _ZR)],
                    cnt_hbm.at[c, pl.ds(s * _ZR, _ZR)])


@functools.cache
def _get_sc_call():
    return functools.partial(
        pl.kernel,
        out_type=[
            jax.ShapeDtypeStruct((_N, _DP), jnp.float32),
            jax.ShapeDtypeStruct((_NC, _K, _CW), jnp.float32),
        ],
        mesh=plsc.VectorSubcoreMesh(core_axis_name="c", subcore_axis_name="s",
                                    num_cores=_NC, num_subcores=_NS),
        scratch_types=[
            pltpu.VMEM((_NCH, _HC), jnp.int32),
            [pltpu.VMEM((_HC, _DP), jnp.float32) for _ in range(_NCH)],
            pltpu.VMEM((_HC, _CW), jnp.float32),
            pltpu.VMEM_SHARED((_K, _CW), jnp.float32),
            pltpu.SemaphoreType.DMA,
        ],
    )(_sc_gather_hist)


def _scalar_body(cnt_ref, md_ref, vq_ref, ppl_ref):
    csum = jnp.sum(cnt_ref[...], axis=0)
    p = csum * (1.0 / _N)
    ent = -jnp.sum(p * jnp.log(p + 1e-10))
    ppl_ref[0, 0] = jnp.exp(ent)
    sl = jnp.sum(md_ref[...])
    cl = sl * (1.0 / (_N * _D))
    vq_ref[0, 0] = cl + 0.25 * cl


def _run_scalars(cnt4, md2):
    return pl.pallas_call(
        _scalar_body,
        out_specs=[
            pl.BlockSpec(memory_space=pltpu.SMEM),
            pl.BlockSpec(memory_space=pltpu.SMEM),
        ],
        out_shape=[
            jax.ShapeDtypeStruct((1, 1), jnp.float32),
            jax.ShapeDtypeStruct((1, 1), jnp.float32),
        ],
    )(cnt4, md2)


def kernel(inputs, W_proj, b_proj, emb):
    inp2d = inputs.reshape(_N, _DIN)
    b2d = b_proj.reshape(1, _D)
    idx3, md3 = _run_proj_argmin(inp2d, W_proj, b2d, emb)
    idx1d = idx3.reshape(_N)
    embp = jnp.concatenate(
        [emb, jnp.zeros((_K, _DP - _D), jnp.float32)], axis=1)
    zeros_hbm = jnp.zeros((_ZR, _CW), jnp.float32)
    ones_hbm = jnp.ones((_HC, _CW), jnp.float32)
    qp, cnt = _get_sc_call()(embp, idx1d, zeros_hbm, ones_hbm)
    q2d = qp[:, :_D]
    cnt4 = cnt[:, :, 0].reshape(_NC, _K // 128, 128)
    md2 = md3.reshape(_N // 128, 128)
    vq, ppl = _run_scalars(cnt4, md2)
    quantized_st = q2d.reshape(_B, _T, _D)
    tokens = idx1d.reshape(_B, _T)
    return quantized_st, tokens, vq[0, 0], ppl[0, 0]


# speedup vs baseline: 1.1546x; 1.0127x over previous
import functools

import jax
import jax.numpy as jnp
from jax import lax
from jax.experimental import pallas as pl
from jax.experimental.pallas import tpu as pltpu
from jax.experimental.pallas import tpu_sc as plsc

_B, _T, _DIN, _D, _K = 16, 512, 768, 64, 8192
_N = _B * _T
_TB = 512
_KB = 2048
_CHUNK = 2048
_G1 = _N // _TB


def _proj_argmin_body(inp_ref, w_ref, b_ref, emb_ref, idx_ref, md_ref):
    x = lax.dot_general(inp_ref[...].astype(jnp.bfloat16),
                        w_ref[...].astype(jnp.bfloat16),
                        (((1,), (1,)), ((), ())),
                        preferred_element_type=jnp.float32)
    x = x + b_ref[...]
    xsq = jnp.sum(x * x, axis=1, keepdims=True)
    x2b = (2.0 * x).astype(jnp.bfloat16)
    iota = lax.broadcasted_iota(jnp.int32, (_TB, _KB), 1)
    acc_v = None
    acc_i = None
    md = None
    for j in range(_K // _KB):
        e = emb_ref[j * _KB:(j + 1) * _KB, :]
        esq = jnp.sum(e * e, axis=1)
        m = lax.dot_general(x2b, e.astype(jnp.bfloat16),
                            (((1,), (1,)), ((), ())),
                            preferred_element_type=jnp.float32)
        d = (xsq + esq[None, :]) - m
        bmin = jnp.min(d, axis=1)
        bidx = jnp.min(jnp.where(d == bmin[:, None], iota, _K), axis=1) + (j * _KB)
        if acc_v is None:
            acc_v, acc_i, md = bmin, bidx, bmin
        else:
            upd = bmin < acc_v
            acc_v = jnp.where(upd, bmin, acc_v)
            acc_i = jnp.where(upd, bidx, acc_i)
            md = jnp.where(upd, bmin, md)
        if (j + 1) % (_CHUNK // _KB) == 0:
            acc_v = acc_v.astype(jnp.bfloat16).astype(jnp.float32)
    idx_ref[0, 0, :] = acc_i
    md_ref[0, 0, :] = md


def _run_proj_argmin(inp2d, w, b2d, emb):
    return pl.pallas_call(
        _proj_argmin_body,
        grid=(_G1,),
        in_specs=[
            pl.BlockSpec((_TB, _DIN), lambda i: (i, 0)),
            pl.BlockSpec((_D, _DIN), lambda i: (0, 0)),
            pl.BlockSpec((1, _D), lambda i: (0, 0)),
            pl.BlockSpec((_K, _D), lambda i: (0, 0)),
        ],
        out_specs=[
            pl.BlockSpec((1, 1, _TB), lambda i: (i, 0, 0)),
            pl.BlockSpec((1, 1, _TB), lambda i: (i, 0, 0)),
        ],
        out_shape=[
            jax.ShapeDtypeStruct((_G1, 1, _TB), jnp.int32),
            jax.ShapeDtypeStruct((_G1, 1, _TB), jnp.float32),
        ],
    )(inp2d, w, b2d, emb)


try:
    _INFO = plsc.get_sparse_core_info()
    _NC, _NS, _L = _INFO.num_cores, _INFO.num_subcores, _INFO.num_lanes
except ValueError:
    _NC, _NS, _L = 2, 16, 16
_NW = _NC * _NS
_BPW = _N // _NW
_HC = 128
_NCH = _BPW // _HC
_DP = 128
_CW = 128
_ZR = _K // _NS


def _sc_gather_hist(emb_hbm, idx_hbm, zeros_hbm, ones_hbm, q_hbm, cnt_hbm,
                    idx_v, rows_v, ones_v, shared, sem):
    c = lax.axis_index("c")
    s = lax.axis_index("s")
    wid = s * _NC + c
    base = wid * _BPW
    cps = []
    for j in range(_NCH):
        pltpu.sync_copy(idx_hbm.at[pl.ds(base + j * _HC, _HC)], idx_v.at[j])
        cps.append(pltpu.async_copy(emb_hbm.at[idx_v.at[j]], rows_v[j], sem))
    pltpu.sync_copy(ones_hbm, ones_v)
    pltpu.sync_copy(zeros_hbm, shared.at[pl.ds(s * _ZR, _ZR)])
    plsc.subcore_barrier()
    for j in range(_NCH):
        pltpu.sync_copy(ones_v, shared.at[idx_v.at[j]], add=True)
    for j in range(_NCH):
        cps[j].wait()
        pltpu.sync_copy(rows_v[j], q_hbm.at[pl.ds(base + j * _HC, _HC)])
    plsc.subcore_barrier()
    pltpu.sync_copy(shared.at[pl.ds(s * _ZR, _ZR)],
                    cnt_hbm.at[c, pl.ds(s * _ZR, _ZR)])


@functools.cache
def _get_sc_call():
    return functools.partial(
        pl.kernel,
        out_type=[
            jax.ShapeDtypeStruct((_N, _DP), jnp.float32),
            jax.ShapeDtypeStruct((_NC, _K, _CW), jnp.float32),
        ],
        mesh=plsc.VectorSubcoreMesh(core_axis_name="c", subcore_axis_name="s",
                                    num_cores=_NC, num_subcores=_NS),
        scratch_types=[
            pltpu.VMEM((_NCH, _HC), jnp.int32),
            [pltpu.VMEM((_HC, _DP), jnp.float32) for _ in range(_NCH)],
            pltpu.VMEM((_HC, _CW), jnp.float32),
            pltpu.VMEM_SHARED((_K, _CW), jnp.float32),
            pltpu.SemaphoreType.DMA,
        ],
    )(_sc_gather_hist)


def _scalar_body(cnt_ref, md_ref, vq_ref, ppl_ref):
    csum = jnp.sum(cnt_ref[...], axis=(0, 2))
    p = csum * (1.0 / (_N * _CW))
    ent = -jnp.sum(p * jnp.log(p + 1e-10))
    ppl_ref[0, 0] = jnp.exp(ent)
    sl = jnp.sum(md_ref[...])
    cl = sl * (1.0 / (_N * _D))
    vq_ref[0, 0] = cl + 0.25 * cl


def _run_scalars(cnt4, md2):
    return pl.pallas_call(
        _scalar_body,
        out_specs=[
            pl.BlockSpec(memory_space=pltpu.SMEM),
            pl.BlockSpec(memory_space=pltpu.SMEM),
        ],
        out_shape=[
            jax.ShapeDtypeStruct((1, 1), jnp.float32),
            jax.ShapeDtypeStruct((1, 1), jnp.float32),
        ],
    )(cnt4, md2)


def kernel(inputs, W_proj, b_proj, emb):
    inp2d = inputs.reshape(_N, _DIN)
    b2d = b_proj.reshape(1, _D)
    idx3, md3 = _run_proj_argmin(inp2d, W_proj, b2d, emb)
    idx1d = idx3.reshape(_N)
    embp = jnp.concatenate(
        [emb, jnp.zeros((_K, _DP - _D), jnp.float32)], axis=1)
    zeros_hbm = jnp.zeros((_ZR, _CW), jnp.float32)
    ones_hbm = jnp.ones((_HC, _CW), jnp.float32)
    qp, cnt = _get_sc_call()(embp, idx1d, zeros_hbm, ones_hbm)
    q2d = qp[:, :_D]

    md2 = md3.reshape(_N // 128, 128)
    vq, ppl = _run_scalars(cnt, md2)
    quantized_st = q2d.reshape(_B, _T, _D)
    tokens = idx1d.reshape(_B, _T)
    return quantized_st, tokens, vq[0, 0], ppl[0, 0]
